# baseline (speedup 1.0000x reference)
import functools

import jax
import jax.numpy as jnp
from jax import lax
from jax.experimental import pallas as pl
from jax.experimental.pallas import tpu as pltpu
from jax.experimental.pallas import tpu_sc as plsc

EMBED_DIM = 128
NUM_CORES = 2
NUM_SUBCORES = 16
NUM_WORKERS = NUM_CORES * NUM_SUBCORES
K = 8


def _gather_body(b_per_w, seq, idx_hbm, table_hbm, out_hbm, idx_v, rows_v, *sems):
    gsems, ssems = sems[:K], sems[K:]
    wid = lax.axis_index("s") * NUM_CORES + lax.axis_index("c")
    base = wid * b_per_w
    pltpu.sync_copy(idx_hbm.at[pl.ds(base, b_per_w)], idx_v)

    def gstart(j, b):
        pltpu.async_copy(table_hbm.at[idx_v.at[j]], rows_v.at[b], gsems[b])

    def gwait(b):
        pltpu.make_async_copy(
            table_hbm.at[idx_v.at[0]], rows_v.at[b], gsems[b]
        ).wait()

    def sstart(j, b):
        pltpu.async_copy(rows_v.at[b], out_hbm.at[base + j], ssems[b])

    def swait(b):
        pltpu.make_async_copy(rows_v.at[b], out_hbm.at[0], ssems[b]).wait()

    for b in range(K):
        gstart(b, b)

    n_outer = b_per_w // K

    def outer(i, carry):
        j0 = i * K
        for b in range(K):
            gwait(b)
            sstart(j0 + b, b)
        for b in range(K):
            swait(b)
            gstart(j0 + K + b, b)
        return carry

    lax.fori_loop(0, n_outer - 1, outer, 0)

    j0 = (n_outer - 1) * K
    for b in range(K):
        gwait(b)
        sstart(j0 + b, b)
    for b in range(K):
        swait(b)


def kernel(x, table):
    B, S = x.shape
    assert B % NUM_WORKERS == 0
    b_per_w = B // NUM_WORKERS
    assert b_per_w % K == 0
    idx = x.astype(jnp.int32)

    mesh = plsc.VectorSubcoreMesh(core_axis_name="c", subcore_axis_name="s")
    k = pl.kernel(
        functools.partial(_gather_body, b_per_w, S),
        out_type=jax.ShapeDtypeStruct((B, S, EMBED_DIM), jnp.float32),
        mesh=mesh,
        scratch_types=[
            pltpu.VMEM((b_per_w, S), jnp.int32),
            pltpu.VMEM((K, S, EMBED_DIM), jnp.float32),
        ]
        + [pltpu.SemaphoreType.DMA] * (2 * K),
        compiler_params=pltpu.CompilerParams(use_tc_tiling_on_sc=True),
    )
    return k(idx, table)


# speedup vs baseline: 1.7219x; 1.7219x over previous
import functools

import jax
import jax.numpy as jnp
from jax import lax
from jax.experimental import pallas as pl
from jax.experimental.pallas import tpu as pltpu
from jax.experimental.pallas import tpu_sc as plsc

EMBED_DIM = 128
NUM_CORES = 2
NUM_SUBCORES = 16
NUM_WORKERS = NUM_CORES * NUM_SUBCORES
U = 64
K = 10


def _gather_body(b_per_w, seq, idx_hbm, table_hbm, out_hbm, idx_v, bufs, *sems):
    gsems, ssems = sems[:K], sems[K:]
    wid = lax.axis_index("s") * NUM_CORES + lax.axis_index("c")
    b0 = wid * b_per_w
    pltpu.sync_copy(idx_hbm.at[wid], idx_v)

    def gstart(u, b):
        pltpu.async_copy(table_hbm.at[idx_v.at[u]], bufs.at[b], gsems[b])

    def gwait(b):
        pltpu.make_async_copy(table_hbm.at[idx_v.at[0]], bufs.at[b], gsems[b]).wait()

    def sstart(j, h, b):
        pltpu.async_copy(
            bufs.at[b], out_hbm.at[j, pl.ds(b0 + h * U, U)], ssems[b]
        )

    def swait(b):
        pltpu.make_async_copy(
            bufs.at[b], out_hbm.at[0, pl.ds(b0, U)], ssems[b]
        ).wait()

    for b in range(K):
        gstart(b, b)

    n_units = 2 * seq
    n_full = n_units // K

    def outer(i, carry):
        for b in range(K):
            gwait(b)
            sstart(i * (K // 2) + b // 2, b % 2, b)
        for b in range(K):
            swait(b)
            gstart((i + 1) * K + b, b)
        return carry

    lax.fori_loop(0, n_full - 1, outer, 0)

    for b in range(K):
        gwait(b)
        sstart((n_full - 1) * (K // 2) + b // 2, b % 2, b)
    for b in range(K):
        swait(b)


def kernel(x, table):
    B, S = x.shape
    assert B % NUM_WORKERS == 0
    b_per_w = B // NUM_WORKERS
    assert b_per_w == 2 * U and (2 * S) % K == 0 and K % 2 == 0
    idx = (
        x.astype(jnp.int32)
        .T.reshape(S, NUM_WORKERS, 2, U)
        .transpose(1, 0, 2, 3)
        .reshape(NUM_WORKERS, 2 * S, U)
    )

    mesh = plsc.VectorSubcoreMesh(core_axis_name="c", subcore_axis_name="s")
    k = pl.kernel(
        functools.partial(_gather_body, b_per_w, S),
        out_type=jax.ShapeDtypeStruct((S, B, EMBED_DIM), jnp.float32),
        mesh=mesh,
        scratch_types=[
            pltpu.VMEM((2 * S, U), jnp.int32),
            pltpu.VMEM((K, U, EMBED_DIM), jnp.float32),
        ]
        + [pltpu.SemaphoreType.DMA] * (2 * K),
    )
    out = k(idx, table)
    return jnp.transpose(out, (1, 0, 2))


# speedup vs baseline: 1.7549x; 1.0192x over previous
import functools

import jax
import jax.numpy as jnp
from jax import lax
from jax.experimental import pallas as pl
from jax.experimental.pallas import tpu as pltpu
from jax.experimental.pallas import tpu_sc as plsc

EMBED_DIM = 128
NUM_CORES = 2
NUM_SUBCORES = 16
NUM_WORKERS = NUM_CORES * NUM_SUBCORES
K = 7


def _gather_body(b_per_w, seq, idx_hbm, table_hbm, out_hbm, idx_v, bufs, *sems):
    gsems, ssems = sems[:K], sems[K:]
    wid = lax.axis_index("s") * NUM_CORES + lax.axis_index("c")
    b0 = wid * b_per_w
    pltpu.sync_copy(idx_hbm.at[:, pl.ds(b0, b_per_w)], idx_v)

    def gstart(j, b):
        pltpu.async_copy(table_hbm.at[idx_v.at[j]], bufs.at[b], gsems[b])

    def gwait(b):
        pltpu.make_async_copy(table_hbm.at[idx_v.at[0]], bufs.at[b], gsems[b]).wait()

    def sstart(j, b):
        pltpu.async_copy(bufs.at[b], out_hbm.at[j, pl.ds(b0, b_per_w)], ssems[b])

    def swait(b):
        pltpu.make_async_copy(
            bufs.at[b], out_hbm.at[0, pl.ds(b0, b_per_w)], ssems[b]
        ).wait()

    for b in range(K):
        gstart(b, b)

    n_full = seq // K
    rem = seq - n_full * K

    def outer(i, carry):
        j0 = i * K
        for b in range(K):
            gwait(b)
            sstart(j0 + b, b)
        for b in range(K):
            swait(b)
            gstart(j0 + K + b, b)
        return carry

    lax.fori_loop(0, n_full - 1, outer, 0)

    j0 = (n_full - 1) * K
    for b in range(K):
        gwait(b)
        sstart(j0 + b, b)
    for r in range(rem):
        u = n_full * K + r
        swait(r)
        gstart(u, r)
        gwait(r)
        sstart(u, r)
    for b in range(K):
        swait(b)


def kernel(x, table):
    B, S = x.shape
    assert B % NUM_WORKERS == 0 and S // K >= 1
    b_per_w = B // NUM_WORKERS
    assert b_per_w <= 128
    idx_t = x.astype(jnp.int32).T

    mesh = plsc.VectorSubcoreMesh(core_axis_name="c", subcore_axis_name="s")
    k = pl.kernel(
        functools.partial(_gather_body, b_per_w, S),
        out_type=jax.ShapeDtypeStruct((S, B, EMBED_DIM), jnp.float32),
        mesh=mesh,
        scratch_types=[
            pltpu.VMEM((S, b_per_w), jnp.int32),
            pltpu.VMEM((K, b_per_w, EMBED_DIM), jnp.float32),
        ]
        + [pltpu.SemaphoreType.DMA] * (2 * K),
    )
    out = k(idx_t, table)
    return jnp.transpose(out, (1, 0, 2))


# speedup vs baseline: 1.7705x; 1.0089x over previous
import functools

import jax
import jax.numpy as jnp
from jax import lax
from jax.experimental import pallas as pl
from jax.experimental.pallas import tpu as pltpu
from jax.experimental.pallas import tpu_sc as plsc

EMBED_DIM = 128
K = 7


def _gather_body(num_cores, b_per_w, seq, idx_hbm, table_hbm, out_hbm, idx_v, bufs, *sems):
    gsems, ssems = sems[:K], sems[K:]
    wid = lax.axis_index("s") * num_cores + lax.axis_index("c")
    b0 = wid * b_per_w
    pltpu.sync_copy(idx_hbm.at[:, pl.ds(b0, b_per_w)], idx_v)

    def gstart(j, b):
        pltpu.async_copy(table_hbm.at[idx_v.at[j]], bufs.at[b], gsems[b])

    def gwait(b):
        pltpu.make_async_copy(table_hbm.at[idx_v.at[0]], bufs.at[b], gsems[b]).wait()

    def sstart(j, b):
        pltpu.async_copy(bufs.at[b], out_hbm.at[j, pl.ds(b0, b_per_w)], ssems[b])

    def swait(b):
        pltpu.make_async_copy(
            bufs.at[b], out_hbm.at[0, pl.ds(b0, b_per_w)], ssems[b]
        ).wait()

    for b in range(K):
        gstart(b, b)

    n_full = seq // K
    rem = seq - n_full * K

    def outer(i, carry):
        j0 = i * K
        for b in range(K):
            gwait(b)
            sstart(j0 + b, b)
        for b in range(K):
            swait(b)
            gstart(j0 + K + b, b)
        return carry

    lax.fori_loop(0, n_full - 1, outer, 0)

    j0 = (n_full - 1) * K
    for b in range(K):
        gwait(b)
        sstart(j0 + b, b)
    for r in range(rem):
        u = n_full * K + r
        swait(r)
        gstart(u, r)
        gwait(r)
        sstart(u, r)
    for b in range(K):
        swait(b)


def kernel(x, table):
    B, S = x.shape
    info = plsc.get_sparse_core_info()
    num_workers = info.num_cores * info.num_subcores
    assert B % num_workers == 0 and S // K >= 1
    b_per_w = B // num_workers
    assert b_per_w <= 128
    idx_t = x.astype(jnp.int32).T

    mesh = plsc.VectorSubcoreMesh(core_axis_name="c", subcore_axis_name="s")
    k = pl.kernel(
        functools.partial(_gather_body, info.num_cores, b_per_w, S),
        out_type=jax.ShapeDtypeStruct((S, B, EMBED_DIM), jnp.float32),
        mesh=mesh,
        scratch_types=[
            pltpu.VMEM((S, b_per_w), jnp.int32),
            pltpu.VMEM((K, b_per_w, EMBED_DIM), jnp.float32),
        ]
        + [pltpu.SemaphoreType.DMA] * (2 * K),
    )
    out = k(idx_t, table)
    return jnp.transpose(out, (1, 0, 2))
